# baseline (speedup 1.0000x reference)
import jax
import jax.numpy as jnp
from jax import lax
from jax.experimental import pallas as pl
from jax.experimental.pallas import tpu as pltpu
from jax.experimental.pallas import tpu_sc as plsc

N = 10000
D = 128
E1 = 320000
E2 = 640000
NS = 16
K = 128
CPB = 16
BLKS1 = 10
BLKS2 = 20
E1_PAD = NS * BLKS1 * CPB * K
E2_PAD = NS * BLKS2 * CPB * K
N_PAD = 10240
ROWS_PER_TILE = N_PAD // NS
LAST_ROWS = N - (NS - 1) * ROWS_PER_TILE


def _sc_body(x_hbm, row1, col1, row2, col2, zeros_hbm, out_hbm,
             acc, colb, rowb, rows0, rows1, gsem, ssem):
    c = lax.axis_index("c")
    s = lax.axis_index("s")
    rbase = s * ROWS_PER_TILE
    rows_bufs = (rows0, rows1)

    pltpu.sync_copy(zeros_hbm, acc.at[pl.ds(rbase, ROWS_PER_TILE)])
    plsc.subcore_barrier()

    def edge_loop(row_hbm, col_hbm, n_blocks):
        tile_chunk_base = s * n_blocks * CPB

        def block_body(blk, carry):
            bbase = tile_chunk_base + blk * CPB
            pltpu.sync_copy(col_hbm.at[pl.ds(bbase, CPB)], colb)
            pltpu.sync_copy(row_hbm.at[pl.ds(bbase, CPB)], rowb)

            def gather(j):
                b = j % 2
                return pltpu.async_copy(
                    x_hbm.at[colb.at[j]], rows_bufs[b], gsem.at[b])

            def scatter(j):
                b = j % 2
                return pltpu.async_copy(
                    rows_bufs[b], acc.at[rowb.at[j]], ssem.at[b], add=True)

            g = [None, None]
            sc = [None, None]
            g[0] = gather(0)
            for j in range(CPB):
                b = j % 2
                if j + 1 < CPB:
                    nb = (j + 1) % 2
                    if j >= 1:
                        sc[nb].wait()
                    g[nb] = gather(j + 1)
                g[b].wait()
                sc[b] = scatter(j)
            sc[(CPB - 2) % 2].wait()
            sc[(CPB - 1) % 2].wait()
            return carry

        lax.fori_loop(0, n_blocks, block_body, 0)

    @pl.when(c == 0)
    def _():
        edge_loop(row1, col1, BLKS1)

    @pl.when(c == 1)
    def _():
        edge_loop(row2, col2, BLKS2)

    plsc.subcore_barrier()

    def writeout(col0):
        @pl.when(s < NS - 1)
        def _():
            pltpu.sync_copy(
                acc.at[pl.ds(rbase, ROWS_PER_TILE)],
                out_hbm.at[pl.ds(rbase, ROWS_PER_TILE), pl.ds(col0, D)])

        @pl.when(s == NS - 1)
        def _():
            pltpu.sync_copy(
                acc.at[pl.ds((NS - 1) * ROWS_PER_TILE, LAST_ROWS)],
                out_hbm.at[pl.ds((NS - 1) * ROWS_PER_TILE, LAST_ROWS),
                           pl.ds(col0, D)])

    @pl.when(c == 0)
    def _():
        writeout(0)

    @pl.when(c == 1)
    def _():
        writeout(D)


def _pad_edges(adj, e_pad):
    e = adj.shape[1]
    row = jnp.concatenate(
        [adj[0], jnp.full((e_pad - e,), N_PAD - 1, jnp.int32)]).reshape(-1, K)
    col = jnp.concatenate(
        [adj[1], jnp.zeros((e_pad - e,), jnp.int32)]).reshape(-1, K)
    return row, col


@jax.jit
def kernel(x, adj_t, adj_t2):
    row1, col1 = _pad_edges(adj_t, E1_PAD)
    row2, col2 = _pad_edges(adj_t2, E2_PAD)
    zeros = jnp.zeros((ROWS_PER_TILE, D), jnp.float32)
    mesh = plsc.VectorSubcoreMesh(core_axis_name="c", subcore_axis_name="s")
    f = pl.kernel(
        _sc_body,
        out_type=jax.ShapeDtypeStruct((N, 2 * D), jnp.float32),
        mesh=mesh,
        scratch_types=[
            pltpu.VMEM_SHARED((N_PAD, D), jnp.float32),
            pltpu.VMEM((CPB, K), jnp.int32),
            pltpu.VMEM((CPB, K), jnp.int32),
            pltpu.VMEM((K, D), jnp.float32),
            pltpu.VMEM((K, D), jnp.float32),
            pltpu.SemaphoreType.DMA((2,)),
            pltpu.SemaphoreType.DMA((2,)),
        ],
    )
    return f(x, row1, col1, row2, col2, zeros)


# speedup vs baseline: 2.4169x; 2.4169x over previous
import jax
import jax.numpy as jnp
from jax import lax
from jax.experimental import pallas as pl
from jax.experimental.pallas import tpu as pltpu
from jax.experimental.pallas import tpu_sc as plsc

N = 10000
D = 128
H = D // 2
E1 = 320000
E2 = 640000
NS = 16
K = 64
CPB = 16
BLKS1 = 20
BLKS2 = 40
E1_PAD = NS * BLKS1 * CPB * K
E2_PAD = NS * BLKS2 * CPB * K
N_ACC = 10008
RPT = 632
LAST_ZERO = N_ACC - (NS - 1) * RPT
LAST_OUT = N - (NS - 1) * RPT
DUMMY_ROW = N


def _sc_body(x_lo, x_hi, row1, col1, row2, col2, zeros_hbm,
             o1_lo, o1_hi, o2_lo, o2_hi,
             x_sp, acc1, acc2, colb, rowb, rows0, rows1, gsem, ssem):
    c = lax.axis_index("c")
    s = lax.axis_index("s")
    rbase = s * RPT
    rows_bufs = (rows0, rows1)

    def tile_rows(src, dst, last_rows):
        @pl.when(s < NS - 1)
        def _():
            pltpu.sync_copy(src.at[pl.ds(rbase, RPT)],
                            dst.at[pl.ds(rbase, RPT)])

        @pl.when(s == NS - 1)
        def _():
            pltpu.sync_copy(src.at[pl.ds((NS - 1) * RPT, last_rows)],
                            dst.at[pl.ds((NS - 1) * RPT, last_rows)])

    @pl.when(c == 0)
    def _():
        tile_rows(x_lo, x_sp, LAST_OUT)

    @pl.when(c == 1)
    def _():
        tile_rows(x_hi, x_sp, LAST_OUT)

    tile_rows(zeros_hbm.at[pl.ds(0, N_ACC)], acc1, LAST_ZERO)
    tile_rows(zeros_hbm.at[pl.ds(0, N_ACC)], acc2, LAST_ZERO)
    plsc.subcore_barrier()

    def edge_loop(row_hbm, col_hbm, n_blocks, acc):
        tile_chunk_base = s * n_blocks * CPB

        def block_body(blk, carry):
            bbase = tile_chunk_base + blk * CPB
            pltpu.sync_copy(col_hbm.at[pl.ds(bbase, CPB)], colb)
            pltpu.sync_copy(row_hbm.at[pl.ds(bbase, CPB)], rowb)

            def gather(j):
                b = j % 2
                return pltpu.async_copy(
                    x_sp.at[colb.at[j]], rows_bufs[b], gsem.at[b])

            def scatter(j):
                b = j % 2
                return pltpu.async_copy(
                    rows_bufs[b], acc.at[rowb.at[j]], ssem.at[b], add=True)

            g = [None, None]
            sc = [None, None]
            g[0] = gather(0)
            for j in range(CPB):
                b = j % 2
                if j + 1 < CPB:
                    nb = (j + 1) % 2
                    if j >= 1:
                        sc[nb].wait()
                    g[nb] = gather(j + 1)
                g[b].wait()
                sc[b] = scatter(j)
            sc[(CPB - 2) % 2].wait()
            sc[(CPB - 1) % 2].wait()
            return carry

        lax.fori_loop(0, n_blocks, block_body, 0)

    edge_loop(row1, col1, BLKS1, acc1)
    edge_loop(row2, col2, BLKS2, acc2)

    plsc.subcore_barrier()

    @pl.when(c == 0)
    def _():
        tile_rows(acc1, o1_lo, LAST_OUT)
        tile_rows(acc2, o2_lo, LAST_OUT)

    @pl.when(c == 1)
    def _():
        tile_rows(acc1, o1_hi, LAST_OUT)
        tile_rows(acc2, o2_hi, LAST_OUT)


def _pad_edges(adj, e_pad):
    e = adj.shape[1]
    row = jnp.concatenate(
        [adj[0], jnp.full((e_pad - e,), DUMMY_ROW, jnp.int32)]).reshape(-1, K)
    col = jnp.concatenate(
        [adj[1], jnp.zeros((e_pad - e,), jnp.int32)]).reshape(-1, K)
    return row, col


@jax.jit
def kernel(x, adj_t, adj_t2):
    row1, col1 = _pad_edges(adj_t, E1_PAD)
    row2, col2 = _pad_edges(adj_t2, E2_PAD)
    x_lo, x_hi = x[:, :H], x[:, H:]
    zeros = jnp.zeros((N_ACC, H), jnp.float32)
    mesh = plsc.VectorSubcoreMesh(core_axis_name="c", subcore_axis_name="s")
    half = jax.ShapeDtypeStruct((N, H), jnp.float32)
    f = pl.kernel(
        _sc_body,
        out_type=[half, half, half, half],
        mesh=mesh,
        compiler_params=pltpu.CompilerParams(use_tc_tiling_on_sc=False),
        scratch_types=[
            pltpu.VMEM_SHARED((N, H), jnp.float32),
            pltpu.VMEM_SHARED((N_ACC, H), jnp.float32),
            pltpu.VMEM_SHARED((N_ACC, H), jnp.float32),
            pltpu.VMEM((CPB, K), jnp.int32),
            pltpu.VMEM((CPB, K), jnp.int32),
            pltpu.VMEM((K, H), jnp.float32),
            pltpu.VMEM((K, H), jnp.float32),
            pltpu.SemaphoreType.DMA((2,)),
            pltpu.SemaphoreType.DMA((2,)),
        ],
    )
    o1_lo, o1_hi, o2_lo, o2_hi = f(x_lo, x_hi, row1, col1, row2, col2, zeros)
    return jnp.concatenate([o1_lo, o1_hi, o2_lo, o2_hi], axis=1)
